# Initial kernel scaffold; baseline (speedup 1.0000x reference)
import functools

import jax
import jax.numpy as jnp
from jax import lax
from jax.experimental import pallas as pl
from jax.experimental.pallas import tpu as pltpu
from jax.experimental.pallas import tpu_sc as plsc

D = 128
CHUNK = 128
NUM_CORES = 2
NUM_SUBCORES = 16
NTILES = NUM_CORES * NUM_SUBCORES
LANES = 16


def _sage_aggregate(x_pad, src_pad, dst_pad, with_cnt):
  npad = x_pad.shape[0]
  e_pad = src_pad.shape[0]
  rows_per_tile = npad // NUM_SUBCORES
  zchunks = rows_per_tile // CHUNK
  chunks_per_tile = e_pad // (NTILES * CHUNK)

  out_types = [jax.ShapeDtypeStruct((NUM_CORES, npad, D), jnp.float32)]
  if with_cnt:
    out_types.append(jax.ShapeDtypeStruct((NUM_CORES, npad), jnp.float32))

  scratch = [
      pltpu.VMEM((CHUNK, D), jnp.float32),
      pltpu.VMEM((CHUNK,), jnp.int32),
      pltpu.VMEM((CHUNK,), jnp.int32),
      pltpu.VMEM((CHUNK,), jnp.float32),
      pltpu.VMEM((CHUNK,), jnp.float32),
      pltpu.VMEM_SHARED((npad, D), jnp.float32),
      pltpu.VMEM_SHARED((npad,), jnp.float32),
      pltpu.SemaphoreType.DMA,
  ]
  mesh = plsc.VectorSubcoreMesh(core_axis_name="c", subcore_axis_name="s")

  def body(x_hbm, src_hbm, dst_hbm, *refs):
    if with_cnt:
      (agg_out, cnt_out, rows_v, sidx_v, didx_v, ones_v, zrow_v,
       agg_sh, cnt_sh, sem) = refs
    else:
      (agg_out, rows_v, sidx_v, didx_v, ones_v, zrow_v,
       agg_sh, cnt_sh, sem) = refs
    c = lax.axis_index("c")
    s = lax.axis_index("s")
    wid = s * NUM_CORES + c
    row0 = s * rows_per_tile

    def fill_rows(i, _):
      r = i // (D // LANES)
      col = (i % (D // LANES)) * LANES
      rows_v[r, pl.ds(col, LANES)] = jnp.zeros((LANES,), jnp.float32)
      return 0
    lax.fori_loop(0, CHUNK * (D // LANES), fill_rows, 0)

    def fill_small(i, _):
      ones_v[pl.ds(i * LANES, LANES)] = jnp.ones((LANES,), jnp.float32)
      zrow_v[pl.ds(i * LANES, LANES)] = jnp.zeros((LANES,), jnp.float32)
      return 0
    lax.fori_loop(0, CHUNK // LANES, fill_small, 0)

    def zero_blk(k, _):
      pltpu.sync_copy(rows_v, agg_sh.at[pl.ds(row0 + k * CHUNK, CHUNK)])
      pltpu.sync_copy(zrow_v, cnt_sh.at[pl.ds(row0 + k * CHUNK, CHUNK)])
      return 0
    lax.fori_loop(0, zchunks, zero_blk, 0)
    plsc.subcore_barrier()

    ebase = wid * (chunks_per_tile * CHUNK)

    def edge_step(j, _):
      off = ebase + j * CHUNK
      pltpu.sync_copy(src_hbm.at[pl.ds(off, CHUNK)], sidx_v)
      pltpu.sync_copy(dst_hbm.at[pl.ds(off, CHUNK)], didx_v)
      pltpu.async_copy(x_hbm.at[sidx_v], rows_v, sem).wait()
      pltpu.sync_copy(rows_v, agg_sh.at[didx_v], add=True)
      if with_cnt:
        pltpu.sync_copy(ones_v, cnt_sh.at[didx_v], add=True)
      return 0
    lax.fori_loop(0, chunks_per_tile, edge_step, 0)
    plsc.subcore_barrier()

    def writeback(k, _):
      r = row0 + k * CHUNK
      pltpu.sync_copy(agg_sh.at[pl.ds(r, CHUNK)], rows_v)
      pltpu.sync_copy(rows_v, agg_out.at[c, pl.ds(r, CHUNK)])
      if with_cnt:
        pltpu.sync_copy(cnt_sh.at[pl.ds(r, CHUNK)], zrow_v)
        pltpu.sync_copy(zrow_v, cnt_out.at[c, pl.ds(r, CHUNK)])
      return 0
    lax.fori_loop(0, zchunks, writeback, 0)

  return pl.kernel(
      body,
      out_type=tuple(out_types),
      mesh=mesh,
      scratch_types=scratch,
  )(x_pad, src_pad, dst_pad)


def _sage_linear(agg_parts, cnt_col, x_pad, Wl, b, Wr, relu):
  npad = x_pad.shape[0]
  bn = 2048
  grid = npad // bn

  def body(agg_ref, cnt_ref, x_ref, wl_ref, b_ref, wr_ref, o_ref):
    mean = (agg_ref[0] + agg_ref[1]) / jnp.maximum(cnt_ref[...], 1.0)
    o = (jnp.dot(mean, wl_ref[...], preferred_element_type=jnp.float32)
         + b_ref[...]
         + jnp.dot(x_ref[...], wr_ref[...], preferred_element_type=jnp.float32))
    if relu:
      o = jnp.maximum(o, 0.0)
    o_ref[...] = o

  return pl.pallas_call(
      body,
      grid=(grid,),
      in_specs=[
          pl.BlockSpec((NUM_CORES, bn, D), lambda i: (0, i, 0)),
          pl.BlockSpec((bn, 1), lambda i: (i, 0)),
          pl.BlockSpec((bn, D), lambda i: (i, 0)),
          pl.BlockSpec((D, D), lambda i: (0, 0)),
          pl.BlockSpec((1, D), lambda i: (0, 0)),
          pl.BlockSpec((D, D), lambda i: (0, 0)),
      ],
      out_specs=pl.BlockSpec((bn, D), lambda i: (i, 0)),
      out_shape=jax.ShapeDtypeStruct((npad, D), jnp.float32),
  )(agg_parts, cnt_col, x_pad, Wl, b.reshape(1, D), Wr)


def kernel(x, edge_index, W1l, b1, W1r, W2l, b2, W2r):
  n = x.shape[0]
  e = edge_index.shape[1]
  rows_align = NUM_SUBCORES * CHUNK
  npad = ((n + 1 + rows_align - 1) // rows_align) * rows_align
  e_align = NTILES * CHUNK
  e_pad = ((e + e_align - 1) // e_align) * e_align

  src = edge_index[0].astype(jnp.int32)
  dst = edge_index[1].astype(jnp.int32)
  src_pad = jnp.concatenate([src, jnp.zeros((e_pad - e,), jnp.int32)])
  dst_pad = jnp.concatenate([dst, jnp.full((e_pad - e,), n, jnp.int32)])
  x_pad = jnp.pad(x, ((0, npad - n), (0, 0)))

  agg1, cnt = _sage_aggregate(x_pad, src_pad, dst_pad, True)
  cnt_col = (cnt[0] + cnt[1]).reshape(npad, 1)
  h_pad = _sage_linear(agg1, cnt_col, x_pad, W1l, b1, W1r, relu=True)
  agg2 = _sage_aggregate(h_pad, src_pad, dst_pad, False)
  out_pad = _sage_linear(agg2, cnt_col, h_pad, W2l, b2, W2r, relu=False)
  return out_pad[:n]


# speedup vs baseline: 4.5656x; 4.5656x over previous
import functools

import jax
import jax.numpy as jnp
from jax import lax
from jax.experimental import pallas as pl
from jax.experimental.pallas import tpu as pltpu
from jax.experimental.pallas import tpu_sc as plsc

D = 128
CHUNK = 128
NUM_CORES = 2
NUM_SUBCORES = 16
NTILES = NUM_CORES * NUM_SUBCORES
LANES = 16


def _sage_aggregate(x_pad, src_pad, dst_pad, with_cnt):
  npad = x_pad.shape[0]
  e_pad = src_pad.shape[0]
  rows_per_tile = npad // NUM_SUBCORES
  zchunks = rows_per_tile // CHUNK
  chunks_per_tile = e_pad // (NTILES * CHUNK)

  out_types = [jax.ShapeDtypeStruct((NUM_CORES, npad, D), jnp.float32)]
  if with_cnt:
    out_types.append(jax.ShapeDtypeStruct((NUM_CORES, npad), jnp.float32))

  scratch = [
      pltpu.VMEM((CHUNK, D), jnp.float32),
      pltpu.VMEM((CHUNK,), jnp.int32),
      pltpu.VMEM((CHUNK,), jnp.int32),
      pltpu.VMEM((CHUNK,), jnp.float32),
      pltpu.VMEM((CHUNK,), jnp.float32),
      pltpu.VMEM_SHARED((npad, D), jnp.float32),
      pltpu.VMEM_SHARED((npad,), jnp.float32),
      pltpu.SemaphoreType.DMA,
  ]
  mesh = plsc.VectorSubcoreMesh(core_axis_name="c", subcore_axis_name="s")

  def body(x_hbm, src_hbm, dst_hbm, *refs):
    if with_cnt:
      (agg_out, cnt_out, rows_v, sidx_v, didx_v, ones_v, zrow_v,
       agg_sh, cnt_sh, sem) = refs
    else:
      (agg_out, rows_v, sidx_v, didx_v, ones_v, zrow_v,
       agg_sh, cnt_sh, sem) = refs
    c = lax.axis_index("c")
    s = lax.axis_index("s")
    wid = s * NUM_CORES + c
    row0 = s * rows_per_tile

    def fill_rows(i, _):
      r = i // (D // LANES)
      col = (i % (D // LANES)) * LANES
      rows_v[r, pl.ds(col, LANES)] = jnp.zeros((LANES,), jnp.float32)
      return 0
    lax.fori_loop(0, CHUNK * (D // LANES), fill_rows, 0)

    def fill_small(i, _):
      ones_v[pl.ds(i * LANES, LANES)] = jnp.ones((LANES,), jnp.float32)
      zrow_v[pl.ds(i * LANES, LANES)] = jnp.zeros((LANES,), jnp.float32)
      return 0
    lax.fori_loop(0, CHUNK // LANES, fill_small, 0)

    def zero_blk(k, _):
      pltpu.sync_copy(rows_v, agg_sh.at[pl.ds(row0 + k * CHUNK, CHUNK)])
      pltpu.sync_copy(zrow_v, cnt_sh.at[pl.ds(row0 + k * CHUNK, CHUNK)])
      return 0
    lax.fori_loop(0, zchunks, zero_blk, 0)
    plsc.subcore_barrier()

    ebase = wid * (chunks_per_tile * CHUNK)

    def edge_step(j, _):
      off = ebase + j * CHUNK
      pltpu.sync_copy(src_hbm.at[pl.ds(off, CHUNK)], sidx_v)
      pltpu.sync_copy(dst_hbm.at[pl.ds(off, CHUNK)], didx_v)
      pltpu.async_copy(x_hbm.at[sidx_v], rows_v, sem).wait()
      pltpu.sync_copy(rows_v, agg_sh.at[didx_v], add=True)
      if with_cnt:
        pltpu.sync_copy(ones_v, cnt_sh.at[didx_v], add=True)
      return 0
    lax.fori_loop(0, chunks_per_tile, edge_step, 0)
    plsc.subcore_barrier()

    def writeback(k, _):
      r = row0 + k * CHUNK
      pltpu.sync_copy(agg_sh.at[pl.ds(r, CHUNK)], rows_v)
      pltpu.sync_copy(rows_v, agg_out.at[c, pl.ds(r, CHUNK)])
      if with_cnt:
        pltpu.sync_copy(cnt_sh.at[pl.ds(r, CHUNK)], zrow_v)
        pltpu.sync_copy(zrow_v, cnt_out.at[c, pl.ds(r, CHUNK)])
      return 0
    lax.fori_loop(0, zchunks, writeback, 0)

  res = pl.kernel(
      body,
      out_type=tuple(out_types),
      mesh=mesh,
      scratch_types=scratch,
  )(x_pad, src_pad, dst_pad)
  if not with_cnt and isinstance(res, (tuple, list)):
    res = res[0]
  return res


def _sage_linear(agg_parts, cnt_col, x_pad, Wl, b, Wr, relu):
  npad = x_pad.shape[0]
  bn = 2048
  grid = npad // bn

  def body(agg_ref, cnt_ref, x_ref, wl_ref, b_ref, wr_ref, o_ref):
    mean = (agg_ref[0] + agg_ref[1]) / jnp.maximum(cnt_ref[...], 1.0)
    o = (jnp.dot(mean, wl_ref[...], preferred_element_type=jnp.float32)
         + b_ref[...]
         + jnp.dot(x_ref[...], wr_ref[...], preferred_element_type=jnp.float32))
    if relu:
      o = jnp.maximum(o, 0.0)
    o_ref[...] = o

  return pl.pallas_call(
      body,
      grid=(grid,),
      in_specs=[
          pl.BlockSpec((NUM_CORES, bn, D), lambda i: (0, i, 0)),
          pl.BlockSpec((bn, 1), lambda i: (i, 0)),
          pl.BlockSpec((bn, D), lambda i: (i, 0)),
          pl.BlockSpec((D, D), lambda i: (0, 0)),
          pl.BlockSpec((1, D), lambda i: (0, 0)),
          pl.BlockSpec((D, D), lambda i: (0, 0)),
      ],
      out_specs=pl.BlockSpec((bn, D), lambda i: (i, 0)),
      out_shape=jax.ShapeDtypeStruct((npad, D), jnp.float32),
  )(agg_parts, cnt_col, x_pad, Wl, b.reshape(1, D), Wr)


def kernel(x, edge_index, W1l, b1, W1r, W2l, b2, W2r):
  n = x.shape[0]
  e = edge_index.shape[1]
  rows_align = NUM_SUBCORES * CHUNK
  npad = ((n + 1 + rows_align - 1) // rows_align) * rows_align
  e_align = NTILES * CHUNK
  e_pad = ((e + e_align - 1) // e_align) * e_align

  src = edge_index[0].astype(jnp.int32)
  dst = edge_index[1].astype(jnp.int32)
  src_pad = jnp.concatenate([src, jnp.zeros((e_pad - e,), jnp.int32)])
  dst_pad = jnp.concatenate([dst, jnp.full((e_pad - e,), n, jnp.int32)])
  x_pad = jnp.pad(x, ((0, npad - n), (0, 0)))

  agg1, cnt = _sage_aggregate(x_pad, src_pad, dst_pad, True)
  cnt_col = (cnt[0] + cnt[1]).reshape(npad, 1)
  h_pad = _sage_linear(agg1, cnt_col, x_pad, W1l, b1, W1r, relu=True)
  agg2 = _sage_aggregate(h_pad, src_pad, dst_pad, False)
  out_pad = _sage_linear(agg2, cnt_col, h_pad, W2l, b2, W2r, relu=False)
  return out_pad[:n]
